# Initial kernel scaffold; baseline (speedup 1.0000x reference)
import jax
import jax.numpy as jnp
from jax.experimental import pallas as pl


def kernel(loss, labels, bin_weights):
    raise NotImplementedError("write your pallas kernel here")


# speedup vs baseline: 4.5710x; 4.5710x over previous
import jax
import jax.numpy as jnp
from jax.experimental import pallas as pl
from jax.experimental.pallas import tpu as pltpu

ROWS = 16384
COLS = 100
NUM_BINS = 100
BLK = 2048


def _body(loss_ref, labels_ref, bw_ref, out_ref):
    labels = labels_ref[...]
    s = jnp.sum(labels, axis=1)
    m = s / COLS
    idx = jnp.clip((m * NUM_BINS).astype(jnp.int32), 0, NUM_BINS - 1)
    bins = jax.lax.broadcasted_iota(jnp.int32, (BLK, NUM_BINS), 1)
    onehot = (idx[:, None] == bins)
    w = jnp.sum(jnp.where(onehot, bw_ref[...], 0.0), axis=1)
    lsum = jnp.sum(loss_ref[...], axis=1)
    partial = jnp.sum(lsum * w).reshape(1, 1)

    @pl.when(pl.program_id(0) == 0)
    def _():
        out_ref[...] = jnp.zeros((1, 1), jnp.float32)

    out_ref[...] += partial


def kernel(loss, labels, bin_weights):
    grid = (ROWS // BLK,)
    out = pl.pallas_call(
        _body,
        grid=grid,
        in_specs=[
            pl.BlockSpec((BLK, COLS), lambda i: (i, 0)),
            pl.BlockSpec((BLK, COLS), lambda i: (i, 0)),
            pl.BlockSpec((1, NUM_BINS), lambda i: (0, 0)),
        ],
        out_specs=pl.BlockSpec((1, 1), lambda i: (0, 0)),
        out_shape=jax.ShapeDtypeStruct((1, 1), jnp.float32),
    )(loss, labels, bin_weights.reshape(1, NUM_BINS))
    return out[0, 0] * (1.0 / (ROWS * COLS))
